# baseline (speedup 1.0000x reference)
import functools

import jax
import jax.numpy as jnp
from jax import lax
from jax.experimental import pallas as pl
from jax.experimental.pallas import tpu as pltpu
from jax.experimental.pallas import tpu_sc as plsc


DIM = 64
CHUNK = 128
NROW = 4
NTR = 2


def _make_gather(num_workers: int, seq: int, btiles: int):
  mesh = plsc.VectorSubcoreMesh(core_axis_name="c", subcore_axis_name="s")
  n_chunks = seq * btiles
  cpw = n_chunks // num_workers
  assert cpw * num_workers == n_chunks

  @functools.partial(
      pl.kernel,
      out_type=jax.ShapeDtypeStruct((seq, DIM // 8, btiles, 8, CHUNK),
                                    jnp.float32),
      mesh=mesh,
      scratch_types=[
          pltpu.VMEM((cpw, CHUNK), jnp.int32),
          pltpu.VMEM((NROW, CHUNK, DIM), jnp.float32),
          pltpu.VMEM((NTR, DIM // 8, 8, CHUNK), jnp.float32),
          pltpu.SemaphoreType.DMA((NROW,)),
          pltpu.SemaphoreType.DMA((NTR,)),
      ],
      compiler_params=pltpu.CompilerParams(
          use_tc_tiling_on_sc=False, needs_layout_passes=False
      ),
  )
  def gather_kernel(ids_hbm, table_hbm, out_hbm, idx_v, rows_v, tr_v,
                    gsem, ssem):
    num_cores = lax.axis_size("c")
    wid = lax.axis_index("s") * num_cores + lax.axis_index("c")
    c0 = wid * cpw

    def out_block(c):
      return out_hbm.at[c // btiles, :, c % btiles]

    pltpu.sync_copy(ids_hbm.at[pl.ds(c0, cpw)], idx_v)

    for j in range(2):
      pltpu.async_copy(table_hbm.at[idx_v.at[j]], rows_v.at[j], gsem.at[j])

    row_sel = [jax.lax.iota(jnp.int32, 16) + 16 * g for g in range(8)]

    @pl.loop(0, cpw, step=NROW)
    def _(j0):
      for u in range(NROW):
        cj = j0 + u
        b = u % NROW
        tb = u % NTR

        pltpu.make_async_copy(
            table_hbm.at[idx_v.at[cj]], rows_v.at[b], gsem.at[b]
        ).wait()

        @pl.when(cj + 2 < cpw)
        def _():
          pltpu.async_copy(
              table_hbm.at[idx_v.at[cj + 2]],
              rows_v.at[(u + 2) % NROW],
              gsem.at[(u + 2) % NROW],
          )

        @pl.when(cj >= NTR)
        def _():
          pltpu.make_async_copy(
              tr_v.at[tb], out_block(c0 + cj - NTR), ssem.at[tb]
          ).wait()

        for d in range(DIM):
          col = jnp.full((16,), d, jnp.int32)
          for g in range(8):
            vals = plsc.load_gather(rows_v.at[b], [row_sel[g], col])
            tr_v[tb, d // 8, d % 8, pl.ds(16 * g, 16)] = vals

        pltpu.async_copy(tr_v.at[tb], out_block(c0 + cj), ssem.at[tb])

    for t in range(NTR):
      cj = cpw - NTR + t
      pltpu.make_async_copy(
          tr_v.at[cj % NTR], out_block(c0 + cj), ssem.at[cj % NTR]
      ).wait()

  return gather_kernel


def kernel(input_ids, word_table):
  batch, seq = input_ids.shape
  assert batch % CHUNK == 0
  btiles = batch // CHUNK
  info = plsc.get_sparse_core_info()
  num_workers = info.num_cores * info.num_subcores

  ids_prep = input_ids.T.astype(jnp.int32).reshape(seq * btiles, CHUNK)
  out5d = _make_gather(num_workers, seq, btiles)(ids_prep, word_table)
  out = out5d.transpose(2, 4, 0, 1, 3).reshape(batch, seq, DIM)
  return out


# speedup vs baseline: 1.5749x; 1.5749x over previous
import functools

import jax
import jax.numpy as jnp
from jax import lax
from jax.experimental import pallas as pl
from jax.experimental.pallas import tpu as pltpu
from jax.experimental.pallas import tpu_sc as plsc


DIM = 64
CHUNK = 128
NROW = 4
NTR = 2
PSTRIDE = DIM * 2 + 1


def _make_gather(num_workers: int, seq: int, btiles: int):
  mesh = plsc.VectorSubcoreMesh(core_axis_name="c", subcore_axis_name="s")
  n_chunks = seq * btiles
  cpw = n_chunks // num_workers
  assert cpw * num_workers == n_chunks
  assert cpw % NROW == 0

  @functools.partial(
      pl.kernel,
      out_type=jax.ShapeDtypeStruct((seq, 8, btiles, 8, CHUNK), jnp.float32),
      mesh=mesh,
      scratch_types=[
          pltpu.VMEM((cpw, CHUNK), jnp.int32),
          pltpu.VMEM((NROW, CHUNK, DIM), jnp.float32),
          pltpu.VMEM((DIM * PSTRIDE,), jnp.float32),
          pltpu.VMEM((NTR, 8, 8, CHUNK), jnp.float32),
          pltpu.SemaphoreType.DMA((NROW,)),
          pltpu.SemaphoreType.DMA((NTR,)),
      ],
      compiler_params=pltpu.CompilerParams(
          use_tc_tiling_on_sc=False, needs_layout_passes=False
      ),
  )
  def gather_kernel(ids_hbm, table_hbm, out_hbm, idx_v, rows_v, trp_v, tr2_v,
                    gsem, ssem):
    num_cores = lax.axis_size("c")
    wid = lax.axis_index("s") * num_cores + lax.axis_index("c")
    c0 = wid * cpw

    def out_block(c):
      return out_hbm.at[c // btiles, :, c % btiles]

    pltpu.sync_copy(ids_hbm.at[pl.ds(c0, cpw)], idx_v)

    for j in range(2):
      pltpu.async_copy(table_hbm.at[idx_v.at[j]], rows_v.at[j], gsem.at[j])

    lane = lax.iota(jnp.int32, 16)
    lane_p = lane * PSTRIDE

    @pl.loop(0, cpw, step=NROW)
    def _(j0):
      for u in range(NROW):
        cj = j0 + u
        b = u % NROW
        tb = u % NTR

        pltpu.make_async_copy(
            table_hbm.at[idx_v.at[cj]], rows_v.at[b], gsem.at[b]
        ).wait()

        @pl.when(cj + 2 < cpw)
        def _():
          pltpu.async_copy(
              table_hbm.at[idx_v.at[cj + 2]],
              rows_v.at[(u + 2) % NROW],
              gsem.at[(u + 2) % NROW],
          )

        @pl.loop(0, CHUNK, unroll=8)
        def _(bb):
          for dblock in range(DIM // 16):
            vals = rows_v[b, bb, pl.ds(dblock * 16, 16)]
            plsc.store_scatter(
                trp_v, [lane_p + (dblock * 16 * PSTRIDE + bb)], vals
            )

        @pl.when(cj >= NTR)
        def _():
          pltpu.make_async_copy(
              tr2_v.at[tb], out_block(c0 + cj - NTR), ssem.at[tb]
          ).wait()

        @pl.loop(0, DIM, unroll=8)
        def _(d):
          for h in range(CHUNK // 16):
            tr2_v[tb, d // 8, d % 8, pl.ds(h * 16, 16)] = (
                trp_v[pl.ds(d * PSTRIDE + h * 16, 16)]
            )

        pltpu.async_copy(tr2_v.at[tb], out_block(c0 + cj), ssem.at[tb])

    for t in range(NTR):
      cj = cpw - NTR + t
      pltpu.make_async_copy(
          tr2_v.at[cj % NTR], out_block(c0 + cj), ssem.at[cj % NTR]
      ).wait()

  return gather_kernel


def kernel(input_ids, word_table):
  batch, seq = input_ids.shape
  assert batch % CHUNK == 0
  btiles = batch // CHUNK
  info = plsc.get_sparse_core_info()
  num_workers = info.num_cores * info.num_subcores

  ids_prep = input_ids.T.astype(jnp.int32).reshape(seq * btiles, CHUNK)
  out5d = _make_gather(num_workers, seq, btiles)(ids_prep, word_table)
  out = out5d.transpose(2, 4, 0, 1, 3).reshape(batch, seq, DIM)
  return out


# speedup vs baseline: 2.7094x; 1.7203x over previous
import functools

import jax
import jax.numpy as jnp
from jax import lax
from jax.experimental import pallas as pl
from jax.experimental.pallas import tpu as pltpu
from jax.experimental.pallas import tpu_sc as plsc


DIM = 64
CHUNK = 128
NROW = 4
NTR = 2

_DNUMS = lax.GatherDimensionNumbers(
    offset_dims=(), collapsed_slice_dims=(0,), start_index_map=(0,)
)


def _perm(v, idx2d):
  return lax.gather(v, idx2d, _DNUMS, slice_sizes=(1,),
                    mode=lax.GatherScatterMode.PROMISE_IN_BOUNDS)


def _make_gather(num_workers: int, seq: int, btiles: int):
  mesh = plsc.VectorSubcoreMesh(core_axis_name="c", subcore_axis_name="s")
  n_chunks = seq * btiles
  cpw = n_chunks // num_workers
  assert cpw * num_workers == n_chunks
  assert cpw % NROW == 0

  @functools.partial(
      pl.kernel,
      out_type=jax.ShapeDtypeStruct((seq, 8, btiles, 8, CHUNK), jnp.float32),
      mesh=mesh,
      scratch_types=[
          pltpu.VMEM((cpw, CHUNK), jnp.int32),
          pltpu.VMEM((NROW, CHUNK, DIM), jnp.float32),
          pltpu.VMEM((NTR, 8, 8, CHUNK), jnp.float32),
          pltpu.SemaphoreType.DMA((NROW,)),
          pltpu.SemaphoreType.DMA((NTR,)),
      ],
      compiler_params=pltpu.CompilerParams(
          use_tc_tiling_on_sc=False, needs_layout_passes=False
      ),
  )
  def gather_kernel(ids_hbm, table_hbm, out_hbm, idx_v, rows_v, tr2_v,
                    gsem, ssem):
    num_cores = lax.axis_size("c")
    wid = lax.axis_index("s") * num_cores + lax.axis_index("c")
    c0 = wid * cpw

    def out_block(c):
      return out_hbm.at[c // btiles, :, c % btiles]

    pltpu.sync_copy(ids_hbm.at[pl.ds(c0, cpw)], idx_v)

    for j in range(2):
      pltpu.async_copy(table_hbm.at[idx_v.at[j]], rows_v.at[j], gsem.at[j])

    lane = lax.iota(jnp.int32, 16)
    perm_idx = {s: (lane ^ s).reshape(16, 1) for s in (1, 2, 4, 8)}
    sel_mask = {s: (lane & s) == 0 for s in (1, 2, 4, 8)}

    @pl.loop(0, cpw, step=NROW)
    def _(j0):
      for u in range(NROW):
        cj = j0 + u
        b = u % NROW
        tb = u % NTR

        pltpu.make_async_copy(
            table_hbm.at[idx_v.at[cj]], rows_v.at[b], gsem.at[b]
        ).wait()

        @pl.when(cj + 2 < cpw)
        def _():
          pltpu.async_copy(
              table_hbm.at[idx_v.at[cj + 2]],
              rows_v.at[(u + 2) % NROW],
              gsem.at[(u + 2) % NROW],
          )

        @pl.when(cj >= NTR)
        def _():
          pltpu.make_async_copy(
              tr2_v.at[tb], out_block(c0 + cj - NTR), ssem.at[tb]
          ).wait()

        @pl.loop(0, 32)
        def _(blk):
          g16 = (blk // 4) * 16
          d16 = (blk % 4) * 16
          vs = [rows_v[b, g16 + i, pl.ds(d16, 16)] for i in range(16)]
          for s in (1, 2, 4, 8):
            nv = list(vs)
            for i in range(16):
              if i & s == 0:
                j = i | s
                pa = _perm(vs[j], perm_idx[s])
                pb = _perm(vs[i], perm_idx[s])
                nv[i] = jnp.where(sel_mask[s], vs[i], pa)
                nv[j] = jnp.where(sel_mask[s], pb, vs[j])
            vs = nv
          for jj in range(16):
            d = d16 + jj
            tr2_v[tb, d // 8, d % 8, pl.ds(g16, 16)] = vs[jj]

        pltpu.async_copy(tr2_v.at[tb], out_block(c0 + cj), ssem.at[tb])

    for t in range(NTR):
      cj = cpw - NTR + t
      pltpu.make_async_copy(
          tr2_v.at[cj % NTR], out_block(c0 + cj), ssem.at[cj % NTR]
      ).wait()

  return gather_kernel


def kernel(input_ids, word_table):
  batch, seq = input_ids.shape
  assert batch % CHUNK == 0
  btiles = batch // CHUNK
  info = plsc.get_sparse_core_info()
  num_workers = info.num_cores * info.num_subcores

  ids_prep = (input_ids.T.astype(jnp.int32) * 2).reshape(seq * btiles, CHUNK)
  vocab = word_table.shape[0]
  table_pad = jnp.pad(word_table, ((0, 0), (0, 64)))
  table2 = table_pad.reshape(2 * vocab, DIM)
  out5d = _make_gather(num_workers, seq, btiles)(ids_prep, table2)
  out = out5d.transpose(2, 4, 0, 1, 3).reshape(batch, seq, DIM)
  return out
